# baseline (speedup 1.0000x reference)
import math

import jax
import jax.numpy as jnp
from jax import lax
from jax.experimental import pallas as pl
from jax.experimental.pallas import tpu as pltpu

_EXP2_SCALE = 8.0 * math.log2(math.e)

_BLOCK_B = 4


def _sdpa_body(q_ref, k_ref, v_ref, o_ref):
    bb = _BLOCK_B
    i = pl.program_id(0) * bb

    q = q_ref[pl.ds(i, bb)]
    k = k_ref[pl.ds(i, bb)]

    qk = lax.dot_general(
        q, k,
        dimension_numbers=(((2,), (2,)), ((0,), (0,))),
        preferred_element_type=jnp.float32)

    m = jnp.max(qk, axis=-1, keepdims=True)
    xb = (qk - m).astype(jnp.bfloat16)
    p = jnp.exp2(xb * jnp.bfloat16(_EXP2_SCALE))

    pv = lax.dot_general(
        p, v_ref[pl.ds(i, bb)],
        dimension_numbers=(((2,), (1,)), ((0,), (0,))),
        preferred_element_type=jnp.float32)

    denom = pv[..., 64:65]
    o_ref[pl.ds(i, bb)] = (pv[..., 0:64] * (1.0 / denom)).astype(jnp.bfloat16)


def _half_attention(q, k, v_ext):
    Bh, S, D = q.shape[0], q.shape[1], 64
    vmemspec = pl.BlockSpec(memory_space=pltpu.MemorySpace.VMEM)
    return pl.pallas_call(
        _sdpa_body,
        out_shape=jax.ShapeDtypeStruct((Bh, S, D), jnp.bfloat16),
        grid=(Bh // _BLOCK_B,),
        in_specs=[vmemspec, vmemspec, vmemspec],
        out_specs=vmemspec,
        compiler_params=pltpu.CompilerParams(
            dimension_semantics=("arbitrary",)),
    )(q, k, v_ext)


def kernel(query, key, value):
    B, S, D = query.shape

    q = query.astype(jnp.bfloat16)
    k = key.astype(jnp.bfloat16)
    v_ext = jnp.concatenate(
        [value.astype(jnp.bfloat16), jnp.ones((B, S, 1), jnp.bfloat16)],
        axis=-1)

    h = B // 2
    y0 = _half_attention(q[:h], k[:h], v_ext[:h])
    y1 = _half_attention(q[h:], k[h:], v_ext[h:])

    return jnp.concatenate([y0, y1], axis=0).astype(jnp.float32)
